# Initial kernel scaffold; baseline (speedup 1.0000x reference)
import jax
import jax.numpy as jnp
from jax.experimental import pallas as pl


def kernel(inputs, codebook):
    raise NotImplementedError("write your pallas kernel here")


# speedup vs baseline: 1.4750x; 1.4750x over previous
import jax
import jax.numpy as jnp
from jax.experimental import pallas as pl
from jax.experimental.pallas import tpu as pltpu

NUM_EMB = 1024
DIM = 64
COMMIT = 0.25
TILE_N = 512


def _vq_body(x_ref, cb_ref, q_ref, idx_ref, loss_ref, perp_ref,
             counts_ref, lsum_ref):
    step = pl.program_id(0)
    nsteps = pl.num_programs(0)
    x = x_ref[...]
    cb = cb_ref[...]
    x2 = jnp.sum(x * x, axis=1, keepdims=True)
    cb2 = jnp.sum(cb * cb, axis=1)
    xc = jax.lax.dot_general(x, cb, (((1,), (1,)), ((), ())),
                             preferred_element_type=jnp.float32)
    d = x2 - 2.0 * xc + cb2[None, :]
    dmin = jnp.min(d, axis=1, keepdims=True)
    kiota = jax.lax.broadcasted_iota(jnp.int32, d.shape, 1)
    idx = jnp.min(jnp.where(d == dmin, kiota, NUM_EMB), axis=1)
    onehot = (kiota == idx[:, None]).astype(jnp.float32)
    q = jax.lax.dot_general(onehot, cb, (((1,), (0,)), ((), ())),
                            preferred_element_type=jnp.float32)
    q_ref[...] = x + (q - x)
    idx_ref[0, 0, :] = idx
    diff = q - x
    part_loss = jnp.sum(diff * diff)
    part_counts = jnp.sum(onehot, axis=0)[None, :]

    @pl.when(step == 0)
    def _():
        counts_ref[...] = part_counts
        lsum_ref[0] = part_loss

    @pl.when(step != 0)
    def _():
        counts_ref[...] += part_counts
        lsum_ref[0] += part_loss

    @pl.when(step == nsteps - 1)
    def _():
        n_total = nsteps * TILE_N
        p = counts_ref[...] * (1.0 / n_total)
        perp_ref[0, 0] = jnp.exp(-jnp.sum(p * jnp.log(p + 1e-10)))
        loss_ref[0, 0] = (1.0 + COMMIT) * lsum_ref[0] / (n_total * DIM)


def kernel(inputs, codebook):
    flat = inputs.reshape(-1, DIM)
    n = flat.shape[0]
    grid = (n // TILE_N,)
    q, idx3, loss, perp = pl.pallas_call(
        _vq_body,
        grid=grid,
        in_specs=[
            pl.BlockSpec((TILE_N, DIM), lambda i: (i, 0)),
            pl.BlockSpec((NUM_EMB, DIM), lambda i: (0, 0)),
        ],
        out_specs=[
            pl.BlockSpec((TILE_N, DIM), lambda i: (i, 0)),
            pl.BlockSpec((1, 1, TILE_N), lambda i: (i, 0, 0)),
            pl.BlockSpec(memory_space=pltpu.SMEM),
            pl.BlockSpec(memory_space=pltpu.SMEM),
        ],
        out_shape=[
            jax.ShapeDtypeStruct((n, DIM), jnp.float32),
            jax.ShapeDtypeStruct((n // TILE_N, 1, TILE_N), jnp.int32),
            jax.ShapeDtypeStruct((1, 1), jnp.float32),
            jax.ShapeDtypeStruct((1, 1), jnp.float32),
        ],
        scratch_shapes=[
            pltpu.VMEM((1, NUM_EMB), jnp.float32),
            pltpu.SMEM((1,), jnp.float32),
        ],
        compiler_params=pltpu.CompilerParams(
            dimension_semantics=("arbitrary",)),
    )(flat, codebook)
    return (q.reshape(inputs.shape), loss[0, 0], perp[0, 0],
            idx3.reshape(-1))
